# baseline (device time: 463320 ns/iter reference)
import jax
import jax.numpy as jnp
from jax import lax
from jax.experimental import pallas as pl
from jax.experimental.pallas import tpu as pltpu

N_DEV = 8
A_ROWS = 176


def kernel(x, w_mat):
    m_per, k = x.shape
    _, n_per = w_mat.shape
    bc = m_per - A_ROWS

    def body(
        x_ref, w_ref, out_ref,
        bufR1, bufR2, bufR3,
        bufL1, bufL2, bufL3,
        bufZ1, bufZfR, bufZfL,
        stage,
        send_sems, recv_sems, copy_sems,
    ):
        my = lax.axis_index("i")
        left = (my - 1 + N_DEV) % N_DEV
        right = (my + 1) % N_DEV
        partner = (my + 4) % N_DEV

        barrier_sem = pltpu.get_barrier_semaphore()
        for nbr in (left, right, partner):
            pl.semaphore_signal(
                barrier_sem, inc=1,
                device_id=(nbr,), device_id_type=pl.DeviceIdType.MESH,
            )
        pl.semaphore_wait(barrier_sem, 3)

        def rdma(idx, src, dst, dev):
            return pltpu.make_async_remote_copy(
                src_ref=src, dst_ref=dst,
                send_sem=send_sems.at[idx], recv_sem=recv_sems.at[idx],
                device_id=(dev,), device_id_type=pl.DeviceIdType.MESH,
            )

        r1 = rdma(0, x_ref, bufR1, right)
        l1 = rdma(1, x_ref, bufL1, left)
        z1 = rdma(2, x_ref, bufZ1, partner)
        r1.start()
        l1.start()
        z1.start()

        out_ref[pl.ds(my * m_per, m_per), :] = jnp.dot(
            x_ref[...], w_ref[...], preferred_element_type=jnp.float32
        )

        pending = []

        def stage_piece(i, hbm_src, origin, row_off, nrows):
            slot = i % 2
            cp = pltpu.make_async_copy(
                hbm_src, stage.at[slot, pl.ds(0, nrows), :], copy_sems.at[slot]
            )
            cp.start()
            if pending:
                drain_one()
            pending.append((cp, slot, origin, row_off, nrows))

        def drain_one():
            cp, slot, origin, row_off, nrows = pending.pop(0)
            cp.wait()
            out_ref[pl.ds(origin * m_per + row_off, nrows), :] = jnp.dot(
                stage[slot, 0:nrows, :], w_ref[...],
                preferred_element_type=jnp.float32,
            )

        r1.wait_recv()
        r2 = rdma(3, bufR1, bufR2, right)
        r2.start()
        zfr = rdma(4, bufR1.at[pl.ds(A_ROWS, bc), :], bufZfR, partner)
        zfr.start()
        stage_piece(0, bufR1, (my - 1 + N_DEV) % N_DEV, 0, m_per)

        l1.wait_recv()
        l2 = rdma(5, bufL1, bufL2, left)
        l2.start()
        zfl = rdma(6, bufL1.at[pl.ds(A_ROWS, bc), :], bufZfL, partner)
        zfl.start()
        stage_piece(1, bufL1, (my + 1) % N_DEV, 0, m_per)

        z1.wait_recv()
        stage_piece(2, bufZ1, partner, 0, m_per)

        zfr.wait_recv()
        stage_piece(3, bufZfR, (my + 3) % N_DEV, A_ROWS, bc)

        r2.wait_recv()
        r3 = rdma(7, bufR2.at[pl.ds(0, A_ROWS), :], bufR3, right)
        r3.start()
        stage_piece(4, bufR2, (my - 2 + N_DEV) % N_DEV, 0, m_per)

        l2.wait_recv()
        l3 = rdma(8, bufL2.at[pl.ds(0, A_ROWS), :], bufL3, left)
        l3.start()
        stage_piece(5, bufL2, (my + 2) % N_DEV, 0, m_per)

        zfl.wait_recv()
        stage_piece(6, bufZfL, (my - 3 + N_DEV) % N_DEV, A_ROWS, bc)

        r3.wait_recv()
        stage_piece(7, bufR3, (my - 3 + N_DEV) % N_DEV, 0, A_ROWS)

        l3.wait_recv()
        stage_piece(8, bufL3, (my + 3) % N_DEV, 0, A_ROWS)

        while pending:
            drain_one()

        for d in (r1, l1, z1, r2, zfr, l2, zfl, r3, l3):
            d.wait_send()

    full = jax.ShapeDtypeStruct((m_per, k), jnp.float32)
    a_sl = jax.ShapeDtypeStruct((A_ROWS, k), jnp.float32)
    bc_sl = jax.ShapeDtypeStruct((bc, k), jnp.float32)
    any_spec = pl.BlockSpec(memory_space=pltpu.MemorySpace.HBM)
    outs = pl.pallas_call(
        body,
        out_shape=(
            jax.ShapeDtypeStruct((N_DEV * m_per, n_per), jnp.float32),
            full, full, a_sl,
            full, full, a_sl,
            full, bc_sl, bc_sl,
        ),
        in_specs=[
            pl.BlockSpec(memory_space=pltpu.VMEM),
            pl.BlockSpec(memory_space=pltpu.VMEM),
        ],
        out_specs=(
            pl.BlockSpec(memory_space=pltpu.VMEM),
            any_spec, any_spec, any_spec,
            any_spec, any_spec, any_spec,
            any_spec, any_spec, any_spec,
        ),
        scratch_shapes=[
            pltpu.VMEM((2, m_per, k), jnp.float32),
            pltpu.SemaphoreType.DMA((9,)),
            pltpu.SemaphoreType.DMA((9,)),
            pltpu.SemaphoreType.DMA((2,)),
        ],
        compiler_params=pltpu.CompilerParams(
            collective_id=0,
            vmem_limit_bytes=64 * 1024 * 1024,
        ),
    )(x, w_mat)
    return outs[0]


# device time: 275723 ns/iter; 1.6804x vs baseline; 1.6804x over previous
import jax
import jax.numpy as jnp
from jax import lax
from jax.experimental import pallas as pl
from jax.experimental.pallas import tpu as pltpu

N_DEV = 8
A_ROWS = 256


def kernel(x, w_mat):
    m_per, k = x.shape
    _, n_per = w_mat.shape
    bc = m_per - A_ROWS

    def body(
        x_ref, w_ref, out_ref,
        bufR1, bufR2, bufR3,
        bufL1, bufL2, bufL3,
        cbOwn, cbMid, cbFar,
        stage,
        send_sems, recv_sems, copy_sems,
    ):
        my = lax.axis_index("i")

        def dev(i):
            i = i % N_DEV
            return jnp.where(i < 4, i, 11 - i)

        pos = dev(my)
        even = (pos % 2) == 0
        right = dev(pos + 1)
        leftd = dev(pos - 1)
        chordd = dev(pos + jnp.where(even, 3, -3))

        barrier_sem = pltpu.get_barrier_semaphore()
        for nbr in (leftd, right, chordd):
            pl.semaphore_signal(
                barrier_sem, inc=1,
                device_id=(nbr,), device_id_type=pl.DeviceIdType.MESH,
            )
        pl.semaphore_wait(barrier_sem, 3)

        def rdma(idx, src, dst, dev_id):
            return pltpu.make_async_remote_copy(
                src_ref=src, dst_ref=dst,
                send_sem=send_sems.at[idx], recv_sem=recv_sems.at[idx],
                device_id=(dev_id,), device_id_type=pl.DeviceIdType.MESH,
            )

        r1 = rdma(0, x_ref, bufR1, right)
        l1 = rdma(1, x_ref, bufL1, leftd)
        c0 = rdma(2, x_ref.at[pl.ds(A_ROWS, bc), :], cbOwn, chordd)
        r1.start()
        l1.start()
        c0.start()

        out_ref[pl.ds(my * m_per, m_per), :] = jnp.dot(
            x_ref[...], w_ref[...], preferred_element_type=jnp.float32
        )

        pending = []

        def stage_piece(i, hbm_src, origin, row_off, nrows):
            slot = i % 2
            cp = pltpu.make_async_copy(
                hbm_src, stage.at[slot, pl.ds(0, nrows), :], copy_sems.at[slot]
            )
            cp.start()
            if pending:
                drain_one()
            pending.append((cp, slot, origin, row_off, nrows))

        def drain_one():
            cp, slot, origin, row_off, nrows = pending.pop(0)
            cp.wait()
            out_ref[pl.ds(origin * m_per + row_off, nrows), :] = jnp.dot(
                stage[slot, 0:nrows, :], w_ref[...],
                preferred_element_type=jnp.float32,
            )

        c0.wait_recv()
        stage_piece(0, cbOwn, chordd, A_ROWS, bc)

        cmid_e = rdma(4, bufR1, cbMid, chordd)
        cmid_o = rdma(4, bufL1, cbMid, chordd)

        r1.wait_recv()
        r2 = rdma(3, bufR1, bufR2, right)
        r2.start()

        @pl.when(even)
        def _():
            cmid_e.start()

        stage_piece(1, bufR1, dev(pos - 1), 0, m_per)

        l1.wait_recv()
        l2 = rdma(5, bufL1, bufL2, leftd)
        l2.start()

        @pl.when(jnp.logical_not(even))
        def _():
            cmid_o.start()

        stage_piece(2, bufL1, dev(pos + 1), 0, m_per)

        cfar_e = rdma(7, bufR2.at[pl.ds(A_ROWS, bc), :], cbFar, chordd)
        cfar_o = rdma(7, bufL2.at[pl.ds(A_ROWS, bc), :], cbFar, chordd)

        r2.wait_recv()
        r3 = rdma(6, bufR2.at[pl.ds(0, A_ROWS), :], bufR3, right)
        r3.start()

        @pl.when(even)
        def _():
            cfar_e.start()

        stage_piece(3, bufR2, dev(pos - 2), 0, m_per)

        l2.wait_recv()
        l3 = rdma(8, bufL2.at[pl.ds(0, A_ROWS), :], bufL3, leftd)
        l3.start()

        @pl.when(jnp.logical_not(even))
        def _():
            cfar_o.start()

        stage_piece(4, bufL2, dev(pos + 2), 0, m_per)

        cmid_e.wait_recv()
        stage_piece(5, cbMid, dev(pos + 4), 0, m_per)

        r3.wait_recv()
        stage_piece(6, bufR3, dev(pos - 3), 0, A_ROWS)
        l3.wait_recv()
        stage_piece(7, bufL3, dev(pos + 3), 0, A_ROWS)
        cfar_e.wait_recv()
        far_origin = jnp.where(chordd == dev(pos + 3), dev(pos - 3), dev(pos + 3))
        stage_piece(8, cbFar, far_origin, A_ROWS, bc)

        while pending:
            drain_one()

        for d in (r1, l1, c0, r2, l2, r3, l3):
            d.wait_send()
        cmid_e.wait_send()
        cfar_e.wait_send()

    full = jax.ShapeDtypeStruct((m_per, k), jnp.float32)
    part = jax.ShapeDtypeStruct((bc, k), jnp.float32)
    any_spec = pl.BlockSpec(memory_space=pltpu.MemorySpace.HBM)
    outs = pl.pallas_call(
        body,
        out_shape=(
            jax.ShapeDtypeStruct((N_DEV * m_per, n_per), jnp.float32),
            full, full, jax.ShapeDtypeStruct((A_ROWS, k), jnp.float32),
            full, full, jax.ShapeDtypeStruct((A_ROWS, k), jnp.float32),
            part, full, part,
        ),
        in_specs=[
            pl.BlockSpec(memory_space=pltpu.VMEM),
            pl.BlockSpec(memory_space=pltpu.VMEM),
        ],
        out_specs=(
            pl.BlockSpec(memory_space=pltpu.VMEM),
            any_spec, any_spec, any_spec,
            any_spec, any_spec, any_spec,
            any_spec, any_spec, any_spec,
        ),
        scratch_shapes=[
            pltpu.VMEM((2, m_per, k), jnp.float32),
            pltpu.SemaphoreType.DMA((9,)),
            pltpu.SemaphoreType.DMA((9,)),
            pltpu.SemaphoreType.DMA((2,)),
        ],
        compiler_params=pltpu.CompilerParams(
            collective_id=0,
            vmem_limit_bytes=64 * 1024 * 1024,
        ),
    )(x, w_mat)
    return outs[0]


# device time: 272273 ns/iter; 1.7017x vs baseline; 1.0127x over previous
import jax
import jax.numpy as jnp
from jax import lax
from jax.experimental import pallas as pl
from jax.experimental.pallas import tpu as pltpu

N_DEV = 8
A_ROWS = 256


def kernel(x, w_mat):
    m_per, k = x.shape
    _, n_per = w_mat.shape
    bc = m_per - A_ROWS

    def body(
        x_ref, w_ref, out_ref,
        bufR1, bufR2, bufR3,
        bufL1, bufL2, bufL3,
        cbOwn, cbMid, cbFar,
        stage,
        send_sems, recv_sems, copy_sems,
    ):
        my = lax.axis_index("i")

        def dev(i):
            i = i % N_DEV
            return jnp.where(i < 4, i, 11 - i)

        pos = dev(my)
        even = (pos % 2) == 0
        right = dev(pos + 1)
        leftd = dev(pos - 1)
        chordd = dev(pos + jnp.where(even, 3, -3))

        barrier_sem = pltpu.get_barrier_semaphore()
        for nbr in (leftd, right, chordd):
            pl.semaphore_signal(
                barrier_sem, inc=1,
                device_id=(nbr,), device_id_type=pl.DeviceIdType.MESH,
            )
        pl.semaphore_wait(barrier_sem, 3)

        def rdma(idx, src, dst, dev_id):
            return pltpu.make_async_remote_copy(
                src_ref=src, dst_ref=dst,
                send_sem=send_sems.at[idx], recv_sem=recv_sems.at[idx],
                device_id=(dev_id,), device_id_type=pl.DeviceIdType.MESH,
            )

        r1 = rdma(0, x_ref, bufR1, right)
        l1 = rdma(1, x_ref, bufL1, leftd)
        c0 = rdma(2, x_ref.at[pl.ds(A_ROWS, bc), :], cbOwn, chordd)
        r1.start()
        l1.start()
        c0.start()

        out_ref[pl.ds(my * m_per, m_per), :] = jnp.dot(
            x_ref[...], w_ref[...], preferred_element_type=jnp.float32
        )

        pending = []

        def stage_piece(i, hbm_src, origin, row_off, nrows):
            slot = i % 2
            cp = pltpu.make_async_copy(
                hbm_src, stage.at[slot, pl.ds(0, nrows), :], copy_sems.at[slot]
            )
            cp.start()
            if pending:
                drain_one()
            pending.append((cp, slot, origin, row_off, nrows))

        def drain_one():
            cp, slot, origin, row_off, nrows = pending.pop(0)
            cp.wait()
            out_ref[pl.ds(origin * m_per + row_off, nrows), :] = jnp.dot(
                stage[slot, 0:nrows, :], w_ref[...],
                preferred_element_type=jnp.float32,
            )

        c0.wait_recv()
        stage_piece(0, cbOwn, chordd, A_ROWS, bc)
        drain_one()

        cmid_e = rdma(4, bufR1, cbMid, chordd)
        cmid_o = rdma(4, bufL1, cbMid, chordd)

        r1.wait_recv()
        r2 = rdma(3, bufR1, bufR2, right)
        r2.start()

        @pl.when(even)
        def _():
            cmid_e.start()

        l1.wait_recv()
        l2 = rdma(5, bufL1, bufL2, leftd)
        l2.start()

        @pl.when(jnp.logical_not(even))
        def _():
            cmid_o.start()

        stage_piece(1, bufR1, dev(pos - 1), 0, m_per)
        stage_piece(2, bufL1, dev(pos + 1), 0, m_per)
        drain_one()

        cfar_e = rdma(7, bufR2.at[pl.ds(A_ROWS, bc), :], cbFar, chordd)
        cfar_o = rdma(7, bufL2.at[pl.ds(A_ROWS, bc), :], cbFar, chordd)

        r2.wait_recv()
        r3 = rdma(6, bufR2.at[pl.ds(0, A_ROWS), :], bufR3, right)
        r3.start()

        @pl.when(even)
        def _():
            cfar_e.start()

        l2.wait_recv()
        l3 = rdma(8, bufL2.at[pl.ds(0, A_ROWS), :], bufL3, leftd)
        l3.start()

        @pl.when(jnp.logical_not(even))
        def _():
            cfar_o.start()

        stage_piece(3, bufR2, dev(pos - 2), 0, m_per)
        stage_piece(4, bufL2, dev(pos + 2), 0, m_per)
        cmid_e.wait_recv()
        stage_piece(5, cbMid, dev(pos + 4), 0, m_per)
        drain_one()

        r3.wait_recv()
        stage_piece(6, bufR3, dev(pos - 3), 0, A_ROWS)
        l3.wait_recv()
        stage_piece(7, bufL3, dev(pos + 3), 0, A_ROWS)
        cfar_e.wait_recv()
        far_origin = jnp.where(chordd == dev(pos + 3), dev(pos - 3), dev(pos + 3))
        stage_piece(8, cbFar, far_origin, A_ROWS, bc)

        while pending:
            drain_one()

        for d in (r1, l1, c0, r2, l2, r3, l3):
            d.wait_send()
        cmid_e.wait_send()
        cfar_e.wait_send()

    full = jax.ShapeDtypeStruct((m_per, k), jnp.float32)
    part = jax.ShapeDtypeStruct((bc, k), jnp.float32)
    any_spec = pl.BlockSpec(memory_space=pltpu.MemorySpace.HBM)
    outs = pl.pallas_call(
        body,
        out_shape=(
            jax.ShapeDtypeStruct((N_DEV * m_per, n_per), jnp.float32),
            full, full, jax.ShapeDtypeStruct((A_ROWS, k), jnp.float32),
            full, full, jax.ShapeDtypeStruct((A_ROWS, k), jnp.float32),
            part, full, part,
        ),
        in_specs=[
            pl.BlockSpec(memory_space=pltpu.VMEM),
            pl.BlockSpec(memory_space=pltpu.VMEM),
        ],
        out_specs=(
            pl.BlockSpec(memory_space=pltpu.VMEM),
            any_spec, any_spec, any_spec,
            any_spec, any_spec, any_spec,
            any_spec, any_spec, any_spec,
        ),
        scratch_shapes=[
            pltpu.VMEM((2, m_per, k), jnp.float32),
            pltpu.SemaphoreType.DMA((9,)),
            pltpu.SemaphoreType.DMA((9,)),
            pltpu.SemaphoreType.DMA((2,)),
        ],
        compiler_params=pltpu.CompilerParams(
            collective_id=0,
            vmem_limit_bytes=64 * 1024 * 1024,
        ),
    )(x, w_mat)
    return outs[0]
